# baseline (speedup 1.0000x reference)
import functools

import jax
import jax.numpy as jnp
from jax import lax
from jax.experimental import pallas as pl
from jax.experimental.pallas import tpu as pltpu
from jax.experimental.pallas import tpu_sc as plsc

NUM_CLASSES = 1000000
EMBED = 64
BATCH = 4096
SEQ = 200

NC = 2
NS = 16
NW = NC * NS
ROWS_PER_W = BATCH // NW

CHUNKS = ((0, 128), (128, 72))


def _body(x_hbm, tab_hbm, out_hbm, idx_v, rows_v, out_v, sem):
    wid = lax.axis_index("s") * NC + lax.axis_index("c")
    row0 = wid * ROWS_PER_W

    def per_row(r, carry):
        b = row0 + r
        pltpu.sync_copy(x_hbm.at[b], idx_v)
        cps = []
        for (off, n) in CHUNKS:
            cps.append(
                pltpu.async_copy(
                    tab_hbm.at[idx_v.at[pl.ds(off, n)]],
                    rows_v.at[pl.ds(off, n)],
                    sem,
                )
            )
        for cp in cps:
            cp.wait()
        eye = lax.iota(jnp.int32, 16)

        def per_l(l, c2):
            col = jnp.full((16,), l, jnp.int32)
            for eb in range(EMBED // 16):
                v = rows_v[l, pl.ds(eb * 16, 16)]
                plsc.store_scatter(out_v, [eye + (eb * 16), col], v)
            return c2

        lax.fori_loop(0, SEQ, per_l, 0, unroll=2)
        pltpu.sync_copy(out_v, out_hbm.at[b])
        return carry

    lax.fori_loop(0, ROWS_PER_W, per_row, 0)


@jax.jit
def _run(x, embed_table):
    mesh = plsc.VectorSubcoreMesh(
        core_axis_name="c", subcore_axis_name="s", num_cores=NC, num_subcores=NS
    )
    f = pl.kernel(
        _body,
        out_type=jax.ShapeDtypeStruct((BATCH, EMBED, SEQ), jnp.float32),
        mesh=mesh,
        scratch_types=[
            pltpu.VMEM((SEQ,), jnp.int32),
            pltpu.VMEM((SEQ, EMBED), jnp.float32),
            pltpu.VMEM((EMBED, SEQ), jnp.float32),
            pltpu.SemaphoreType.DMA,
        ],
        compiler_params=pltpu.CompilerParams(
            use_tc_tiling_on_sc=False, needs_layout_passes=False
        ),
    )
    return f(x, embed_table)


def kernel(x, embed_table):
    return _run(x, embed_table)


# speedup vs baseline: 1.3750x; 1.3750x over previous
import jax
import jax.numpy as jnp
from jax import lax
from jax.experimental import pallas as pl
from jax.experimental.pallas import tpu as pltpu
from jax.experimental.pallas import tpu_sc as plsc

NUM_CLASSES = 1000000
EMBED = 64
BATCH = 4096
SEQ = 200

NC = 2
NS = 16
NW = NC * NS
ROWS_PER_W = BATCH // NW

CHUNKS = ((0, 128), (128, 72))


def _body(x_hbm, tab_hbm, out_hbm, idx_all, rows2, out2, sg0, sg1, so0, so1):
    wid = lax.axis_index("s") * NC + lax.axis_index("c")
    row0 = wid * ROWS_PER_W

    pltpu.sync_copy(x_hbm.at[pl.ds(row0 * SEQ, ROWS_PER_W * SEQ)], idx_all)

    def start_gather(r, rbuf, sem):
        base = r * SEQ
        for off, n in CHUNKS:
            pltpu.make_async_copy(
                tab_hbm.at[idx_all.at[pl.ds(base + off, n)]],
                rbuf.at[pl.ds(off, n)],
                sem,
            ).start()

    def wait_gather(rbuf, sem):
        pltpu.make_async_copy(tab_hbm.at[pl.ds(0, SEQ)], rbuf, sem).wait()

    eye = lax.iota(jnp.int32, 16)

    def transpose(rbuf, obuf):
        @plsc.parallel_loop(0, SEQ, step=1, unroll=4)
        def _(l):
            col = jnp.full((16,), l, jnp.int32)
            for eb in range(EMBED // 16):
                v = rbuf[l, pl.ds(eb * 16, 16)]
                plsc.store_scatter(obuf, [eye + (eb * 16), col], v)

    def start_store(r, obuf, sem):
        pltpu.make_async_copy(obuf, out_hbm.at[row0 + r], sem).start()

    def wait_store(r, obuf, sem):
        pltpu.make_async_copy(obuf, out_hbm.at[row0 + r], sem).wait()

    start_gather(0, rows2.at[0], sg0)

    def step(k, carry):
        r0 = 2 * k
        r1 = r0 + 1
        start_gather(r1, rows2.at[1], sg1)
        wait_gather(rows2.at[0], sg0)

        @pl.when(k >= 1)
        def _():
            wait_store(r0 - 2, out2.at[0], so0)

        transpose(rows2.at[0], out2.at[0])
        start_store(r0, out2.at[0], so0)

        @pl.when(k < (ROWS_PER_W // 2 - 1))
        def _():
            start_gather(r1 + 1, rows2.at[0], sg0)

        wait_gather(rows2.at[1], sg1)

        @pl.when(k >= 1)
        def _():
            wait_store(r1 - 2, out2.at[1], so1)

        transpose(rows2.at[1], out2.at[1])
        start_store(r1, out2.at[1], so1)
        return carry

    lax.fori_loop(0, ROWS_PER_W // 2, step, 0)
    wait_store(ROWS_PER_W - 2, out2.at[0], so0)
    wait_store(ROWS_PER_W - 1, out2.at[1], so1)


@jax.jit
def _run(x, embed_table):
    mesh = plsc.VectorSubcoreMesh(
        core_axis_name="c", subcore_axis_name="s", num_cores=NC, num_subcores=NS
    )
    f = pl.kernel(
        _body,
        out_type=jax.ShapeDtypeStruct((BATCH, EMBED, SEQ), jnp.float32),
        mesh=mesh,
        scratch_types=[
            pltpu.VMEM((ROWS_PER_W * SEQ,), jnp.int32),
            pltpu.VMEM((2, SEQ, EMBED), jnp.float32),
            pltpu.VMEM((2, EMBED, SEQ), jnp.float32),
            pltpu.SemaphoreType.DMA,
            pltpu.SemaphoreType.DMA,
            pltpu.SemaphoreType.DMA,
            pltpu.SemaphoreType.DMA,
        ],
        compiler_params=pltpu.CompilerParams(
            use_tc_tiling_on_sc=False, needs_layout_passes=False
        ),
    )
    return f(x.reshape(-1), embed_table)


def kernel(x, embed_table):
    return _run(x, embed_table)


# speedup vs baseline: 1.3810x; 1.0044x over previous
import jax
import jax.numpy as jnp
from jax import lax
from jax.experimental import pallas as pl
from jax.experimental.pallas import tpu as pltpu
from jax.experimental.pallas import tpu_sc as plsc

NUM_CLASSES = 1000000
EMBED = 64
BATCH = 4096
SEQ = 200

NC = 2
NS = 16
NW = NC * NS
ROWS_PER_W = BATCH // NW

CHUNKS = ((0, 128), (128, 72))


NG = 4
NO = 2


def _body(x_hbm, tab_hbm, out_hbm, idx_all, rows4, out2, sg0, sg1, sg2, sg3,
          so0, so1):
    wid = lax.axis_index("s") * NC + lax.axis_index("c")
    row0 = wid * ROWS_PER_W
    sg = (sg0, sg1, sg2, sg3)
    so = (so0, so1)

    pltpu.sync_copy(x_hbm.at[pl.ds(row0 * SEQ, ROWS_PER_W * SEQ)], idx_all)

    def start_gather(r, p):
        base = r * SEQ
        for off, n in CHUNKS:
            pltpu.make_async_copy(
                tab_hbm.at[idx_all.at[pl.ds(base + off, n)]],
                rows4.at[p].at[pl.ds(off, n)],
                sg[p],
            ).start()

    def wait_gather(p):
        for off, n in CHUNKS:
            pltpu.make_async_copy(
                tab_hbm.at[pl.ds(0, n)],
                rows4.at[p].at[pl.ds(off, n)],
                sg[p],
            ).wait()

    eye = lax.iota(jnp.int32, 16)

    def transpose(rbuf, obuf):
        @plsc.parallel_loop(0, SEQ, step=1, unroll=4)
        def _(l):
            col = jnp.full((16,), l, jnp.int32)
            for eb in range(EMBED // 16):
                v = rbuf[l, pl.ds(eb * 16, 16)]
                plsc.store_scatter(obuf, [eye + (eb * 16), col], v)

    def start_store(r, q):
        pltpu.make_async_copy(out2.at[q], out_hbm.at[row0 + r], so[q]).start()

    def wait_store(r, q):
        pltpu.make_async_copy(out2.at[q], out_hbm.at[row0 + r], so[q]).wait()

    for r in range(NG - 1):
        start_gather(r, r)

    def step(k, carry):
        for j in range(NG):
            r = NG * k + j
            p = j
            q = j % NO

            @pl.when(r + (NG - 1) < ROWS_PER_W)
            def _():
                start_gather(r + (NG - 1), (j + NG - 1) % NG)

            wait_gather(p)

            @pl.when(r >= NO)
            def _():
                wait_store(r - NO, q)

            transpose(rows4.at[p], out2.at[q])
            start_store(r, q)
        return carry

    lax.fori_loop(0, ROWS_PER_W // NG, step, 0)
    wait_store(ROWS_PER_W - 2, 0)
    wait_store(ROWS_PER_W - 1, 1)


@jax.jit
def _run(x, embed_table):
    mesh = plsc.VectorSubcoreMesh(
        core_axis_name="c", subcore_axis_name="s", num_cores=NC, num_subcores=NS
    )
    f = pl.kernel(
        _body,
        out_type=jax.ShapeDtypeStruct((BATCH, EMBED, SEQ), jnp.float32),
        mesh=mesh,
        scratch_types=[
            pltpu.VMEM((ROWS_PER_W * SEQ,), jnp.int32),
            pltpu.VMEM((NG, SEQ, EMBED), jnp.float32),
            pltpu.VMEM((NO, EMBED, SEQ), jnp.float32),
            pltpu.SemaphoreType.DMA,
            pltpu.SemaphoreType.DMA,
            pltpu.SemaphoreType.DMA,
            pltpu.SemaphoreType.DMA,
            pltpu.SemaphoreType.DMA,
            pltpu.SemaphoreType.DMA,
        ],
        compiler_params=pltpu.CompilerParams(
            use_tc_tiling_on_sc=False, needs_layout_passes=False
        ),
    )
    return f(x.reshape(-1), embed_table)


def kernel(x, embed_table):
    return _run(x, embed_table)
